# Initial kernel scaffold; baseline (speedup 1.0000x reference)
import jax
import jax.numpy as jnp
from jax.experimental import pallas as pl


def kernel(text_tokens, emb_table, proj_w, proj_b):
    raise NotImplementedError("write your pallas kernel here")


# speedup vs baseline: 2.8987x; 2.8987x over previous
import functools

import jax
import jax.numpy as jnp
from jax import lax
from jax.experimental import pallas as pl
from jax.experimental.pallas import tpu as pltpu
from jax.experimental.pallas import tpu_sc as plsc


_GATHER_WINDOW = 32


_NUM_WORKERS = 32


def _sc_gather(table, idx, dep=None):
    n_tok = idx.shape[0]
    dim = table.shape[1]
    b_per_w = n_tok // _NUM_WORKERS
    ch = _GATHER_WINDOW
    n_ch = b_per_w // ch
    mesh = plsc.VectorSubcoreMesh(core_axis_name="c", subcore_axis_name="s")

    @functools.partial(
        pl.kernel,
        out_type=jax.ShapeDtypeStruct((n_tok, dim), table.dtype),
        mesh=mesh,
        scratch_types=[
            pltpu.VMEM((b_per_w,), jnp.int32),
            pltpu.VMEM((2, ch, dim), jnp.float32),
            pltpu.SemaphoreType.DMA,
            pltpu.SemaphoreType.DMA,
            pltpu.SemaphoreType.DMA,
            pltpu.SemaphoreType.DMA,
            pltpu.SemaphoreType.DMA,
        ],
    )
    def gather_kernel(table_hbm, idx_hbm, *rest):
        if dep is None:
            out_hbm, idx_v, rows_v, sem_i, sg0, sg1, so0, so1 = rest
        else:
            _dep_hbm, out_hbm, idx_v, rows_v, sem_i, sg0, sg1, so0, so1 = rest
        wid = lax.axis_index("s") * 2 + lax.axis_index("c")
        base = wid * b_per_w
        pltpu.async_copy(idx_hbm.at[pl.ds(base, b_per_w)], idx_v, sem_i).wait()
        sg = (sg0, sg1)
        so = (so0, so1)

        def g_copy(c, s):
            return pltpu.make_async_copy(
                table_hbm.at[idx_v.at[pl.ds(c * ch, ch)]], rows_v.at[s], sg[s])

        def o_copy(c, s):
            return pltpu.make_async_copy(
                rows_v.at[s], out_hbm.at[pl.ds(base + c * ch, ch)], so[s])

        for c in range(n_ch):
            s = c % 2
            if c >= 2:
                o_copy(c - 2, s).wait()
            g_copy(c, s).start()
            if c >= 1:
                g_copy(c - 1, 1 - s).wait()
                o_copy(c - 1, 1 - s).start()
        last = n_ch - 1
        if last >= 1:
            o_copy(last - 1, (last - 1) % 2).wait()
        g_copy(last, last % 2).wait()
        o_copy(last, last % 2).start()
        o_copy(last, last % 2).wait()

    if dep is None:
        return gather_kernel(table, idx)
    return gather_kernel(table, idx, dep)


_BM = 1024


def _mm_body(a_ref, w_ref, b_ref, o_ref):
    o_ref[...] = (
        jnp.dot(a_ref[...], w_ref[...], preferred_element_type=jnp.float32)
        + b_ref[...]
    )


def _mm_body_acc(acc_ref, a_ref, w_ref, b_ref, o_ref):
    del acc_ref
    _mm_body(a_ref, w_ref, b_ref, o_ref)


def _tc_matmul_chunk(acc, a, w, b2d, base_step, m_total):
    m, k = a.shape
    n = w.shape[1]
    steps = m // _BM
    out_spec = pl.BlockSpec((_BM, n), lambda i: (i + base_step, 0))
    data_specs = [
        pl.BlockSpec((_BM, k), lambda i: (i, 0)),
        pl.BlockSpec((k, n), lambda i: (0, 0)),
        pl.BlockSpec((1, n), lambda i: (0, 0)),
    ]
    out_shape = jax.ShapeDtypeStruct((m_total, n), jnp.float32)
    if acc is None:
        return pl.pallas_call(
            _mm_body,
            grid=(steps,),
            in_specs=data_specs,
            out_specs=out_spec,
            out_shape=out_shape,
        )(a, w, b2d)
    return pl.pallas_call(
        _mm_body_acc,
        grid=(steps,),
        in_specs=[pl.BlockSpec(memory_space=pl.ANY)] + data_specs,
        out_specs=out_spec,
        out_shape=out_shape,
        input_output_aliases={0: 0},
    )(acc, a, w, b2d)


_N_CHUNKS = 1


def kernel(text_tokens, emb_table, proj_w, proj_b):
    bsz, seq = text_tokens.shape
    n_out = proj_w.shape[1]
    tokens = text_tokens.reshape(-1).astype(jnp.int32)
    n_tok = tokens.shape[0]
    csz = n_tok // _N_CHUNKS
    b2d = proj_b.reshape(1, n_out)

    def tok_chunk(i):
        return lax.slice_in_dim(tokens, i * csz, (i + 1) * csz)

    gathered = [_sc_gather(emb_table, tok_chunk(0))]
    acc = None
    for i in range(_N_CHUNKS):
        if i + 1 < _N_CHUNKS:
            gathered.append(
                _sc_gather(emb_table, tok_chunk(i + 1), dep=gathered[i]))
        acc = _tc_matmul_chunk(
            acc, gathered[i], proj_w, b2d, i * (csz // _BM), n_tok)
    return acc.reshape(bsz, seq, n_out)
